# baseline (speedup 1.0000x reference)
import jax
import jax.numpy as jnp
from jax.experimental import pallas as pl
from jax.experimental.pallas import tpu as pltpu

_TOP_K = 2


def _router_kernel(x_ref, w_ref, idx_ref, w_out_ref, aux_ref, imp_ref, load_ref):
    b = pl.program_id(0)
    s = pl.program_id(1)
    nb = pl.num_programs(0)
    ns = pl.num_programs(1)

    x2 = x_ref[0]
    logits = jax.lax.dot_general(
        w_ref[:], x2, (((1,), (1,)), ((), ())),
        preferred_element_type=jnp.float32,
    )

    E, T = logits.shape
    row = jax.lax.broadcasted_iota(jnp.int32, (E, T), 0)

    m = jnp.max(logits, axis=0, keepdims=True)
    eq1 = logits == m
    i1 = jnp.min(jnp.where(eq1, row, E), axis=0, keepdims=True)
    lmask = jnp.where(eq1, -jnp.inf, logits)
    l2 = jnp.max(lmask, axis=0, keepdims=True)
    eq2 = lmask == l2
    i2 = jnp.min(jnp.where(eq2, row, E), axis=0, keepdims=True)

    e2 = jnp.exp(l2 - m)
    w1 = 1.0 / (1.0 + e2)
    idx_ref[:] = jnp.concatenate([i1, i2], axis=0)
    w_out_ref[:] = jnp.concatenate([w1, e2 * w1], axis=0)

    e = jnp.exp(logits - m)
    z = jnp.sum(e, axis=0, keepdims=True)
    probs = e * (1.0 / z)
    imp_tile = jnp.sum(probs, axis=1, keepdims=True)
    sel = jnp.logical_or(eq1, eq2)
    load_tile = jnp.sum(sel.astype(jnp.float32), axis=1, keepdims=True)

    first = jnp.logical_and(b == 0, s == 0)
    last = jnp.logical_and(b == nb - 1, s == ns - 1)

    @pl.when(first)
    def _():
        imp_ref[:] = imp_tile
        load_ref[:] = load_tile

    @pl.when(jnp.logical_not(first))
    def _():
        imp_ref[:] = imp_ref[:] + imp_tile
        load_ref[:] = load_ref[:] + load_tile

    @pl.when(last)
    def _():
        n_tok = jnp.float32(nb * ns * T)
        importance = imp_ref[:] / n_tok
        load = load_ref[:] / (n_tok * _TOP_K)
        aux_ref[:] = jnp.sum(E * importance * load).reshape(1, 1)


def kernel(x, W):
    B, S, H = x.shape
    E = W.shape[0]
    T = 2048
    ns = S // T

    idx_t, w_t, aux = pl.pallas_call(
        _router_kernel,
        grid=(B, ns),
        in_specs=[
            pl.BlockSpec((1, T, H), lambda b, s: (b, s, 0)),
            pl.BlockSpec((E, H), lambda b, s: (0, 0)),
        ],
        out_specs=[
            pl.BlockSpec((_TOP_K, T), lambda b, s: (0, b * (S // T) + s)),
            pl.BlockSpec((_TOP_K, T), lambda b, s: (0, b * (S // T) + s)),
            pl.BlockSpec((1, 1), lambda b, s: (0, 0)),
        ],
        out_shape=[
            jax.ShapeDtypeStruct((_TOP_K, B * S), jnp.int32),
            jax.ShapeDtypeStruct((_TOP_K, B * S), jnp.float32),
            jax.ShapeDtypeStruct((1, 1), jnp.float32),
        ],
        scratch_shapes=[
            pltpu.VMEM((E, 1), jnp.float32),
            pltpu.VMEM((E, 1), jnp.float32),
        ],
        compiler_params=pltpu.CompilerParams(
            dimension_semantics=("arbitrary", "arbitrary"),
        ),
    )(x, W)

    return (
        jnp.moveaxis(idx_t.reshape(_TOP_K, B, S), 0, -1),
        jnp.moveaxis(w_t.reshape(_TOP_K, B, S), 0, -1),
        aux[0, 0],
    )


# speedup vs baseline: 1.0424x; 1.0424x over previous
import jax
import jax.numpy as jnp
from jax.experimental import pallas as pl
from jax.experimental.pallas import tpu as pltpu

_TOP_K = 2


def _router_kernel(x_ref, w_ref, idx_ref, w_out_ref, aux_ref, imp_ref, load_ref):
    b = pl.program_id(0)
    s = pl.program_id(1)
    nb = pl.num_programs(0)
    ns = pl.num_programs(1)

    x2 = x_ref[0]
    logits = jax.lax.dot_general(
        w_ref[:], x2, (((1,), (1,)), ((), ())),
        preferred_element_type=jnp.float32,
    )

    E, T = logits.shape
    row = jax.lax.broadcasted_iota(jnp.int32, (E, T), 0)

    m = jnp.max(logits, axis=0, keepdims=True)
    eq1 = logits == m
    i1 = jnp.min(jnp.where(eq1, row, E), axis=0, keepdims=True)
    lmask = jnp.where(eq1, -jnp.inf, logits)
    l2 = jnp.max(lmask, axis=0, keepdims=True)
    eq2 = lmask == l2
    i2 = jnp.min(jnp.where(eq2, row, E), axis=0, keepdims=True)

    e2 = jnp.exp(l2 - m)
    w1 = 1.0 / (1.0 + e2)
    idx_ref[:] = jnp.concatenate([i1, i2], axis=0)
    w_out_ref[:] = jnp.concatenate([w1, e2 * w1], axis=0)

    e = jnp.exp(logits - m)
    z = jnp.sum(e, axis=0, keepdims=True)
    probs = e * (1.0 / z)
    imp_tile = jnp.sum(probs, axis=1, keepdims=True)
    sel = jnp.logical_or(eq1, eq2)
    load_tile = jnp.sum(sel.astype(jnp.float32), axis=1, keepdims=True)

    first = jnp.logical_and(b == 0, s == 0)
    last = jnp.logical_and(b == nb - 1, s == ns - 1)

    @pl.when(first)
    def _():
        imp_ref[:] = imp_tile
        load_ref[:] = load_tile

    @pl.when(jnp.logical_not(first))
    def _():
        imp_ref[:] = imp_ref[:] + imp_tile
        load_ref[:] = load_ref[:] + load_tile

    @pl.when(last)
    def _():
        n_tok = jnp.float32(nb * ns * T)
        importance = imp_ref[:] / n_tok
        load = load_ref[:] / (n_tok * _TOP_K)
        aux_ref[:] = jnp.sum(E * importance * load).reshape(1, 1)


def kernel(x, W):
    B, S, H = x.shape
    E = W.shape[0]
    T = 8192
    ns = S // T

    idx_t, w_t, aux = pl.pallas_call(
        _router_kernel,
        grid=(B, ns),
        in_specs=[
            pl.BlockSpec((1, T, H), lambda b, s: (b, s, 0)),
            pl.BlockSpec((E, H), lambda b, s: (0, 0)),
        ],
        out_specs=[
            pl.BlockSpec((_TOP_K, T), lambda b, s: (0, b * (S // T) + s)),
            pl.BlockSpec((_TOP_K, T), lambda b, s: (0, b * (S // T) + s)),
            pl.BlockSpec((1, 1), lambda b, s: (0, 0)),
        ],
        out_shape=[
            jax.ShapeDtypeStruct((_TOP_K, B * S), jnp.int32),
            jax.ShapeDtypeStruct((_TOP_K, B * S), jnp.float32),
            jax.ShapeDtypeStruct((1, 1), jnp.float32),
        ],
        scratch_shapes=[
            pltpu.VMEM((E, 1), jnp.float32),
            pltpu.VMEM((E, 1), jnp.float32),
        ],
        compiler_params=pltpu.CompilerParams(
            dimension_semantics=("arbitrary", "arbitrary"),
        ),
    )(x, W)

    return (
        jnp.moveaxis(idx_t.reshape(_TOP_K, B, S), 0, -1),
        jnp.moveaxis(w_t.reshape(_TOP_K, B, S), 0, -1),
        aux[0, 0],
    )


# speedup vs baseline: 1.1092x; 1.0641x over previous
import jax
import jax.numpy as jnp
from jax.experimental import pallas as pl
from jax.experimental.pallas import tpu as pltpu

_TOP_K = 2


def _router_kernel(xa_ref, xb_ref, w_ref, idx_ref, w_out_ref, aux_ref, imp_ref, load_ref):
    b = pl.program_id(0)
    s = pl.program_id(1)
    nb = pl.num_programs(0)
    ns = pl.num_programs(1)

    xa = xa_ref[0]
    xb = xb_ref[0]
    H2 = xa.shape[1]
    logits = jax.lax.dot_general(
        w_ref[:, :H2], xa, (((1,), (1,)), ((), ())),
        preferred_element_type=jnp.float32,
    ) + jax.lax.dot_general(
        w_ref[:, H2:], xb, (((1,), (1,)), ((), ())),
        preferred_element_type=jnp.float32,
    )

    E, T = logits.shape
    row = jax.lax.broadcasted_iota(jnp.int32, (E, T), 0)

    m = jnp.max(logits, axis=0, keepdims=True)
    eq1 = logits == m
    i1 = jnp.min(jnp.where(eq1, row, E), axis=0, keepdims=True)
    lmask = jnp.where(eq1, -jnp.inf, logits)
    l2 = jnp.max(lmask, axis=0, keepdims=True)
    eq2 = lmask == l2
    i2 = jnp.min(jnp.where(eq2, row, E), axis=0, keepdims=True)

    e2 = jnp.exp(l2 - m)
    w1 = 1.0 / (1.0 + e2)
    idx_ref[:] = jnp.concatenate([i1, i2], axis=0)
    w_out_ref[:] = jnp.concatenate([w1, e2 * w1], axis=0)

    e = jnp.exp(logits - m)
    z = jnp.sum(e, axis=0, keepdims=True)
    probs = e * (1.0 / z)
    imp_tile = jnp.sum(probs, axis=1, keepdims=True)
    sel = jnp.logical_or(eq1, eq2)
    load_tile = jnp.sum(sel.astype(jnp.float32), axis=1, keepdims=True)

    first = jnp.logical_and(b == 0, s == 0)
    last = jnp.logical_and(b == nb - 1, s == ns - 1)

    @pl.when(first)
    def _():
        imp_ref[:] = imp_tile
        load_ref[:] = load_tile

    @pl.when(jnp.logical_not(first))
    def _():
        imp_ref[:] = imp_ref[:] + imp_tile
        load_ref[:] = load_ref[:] + load_tile

    @pl.when(last)
    def _():
        n_tok = jnp.float32(nb * ns * T)
        importance = imp_ref[:] / n_tok
        load = load_ref[:] / (n_tok * _TOP_K)
        aux_ref[:] = jnp.sum(E * importance * load).reshape(1, 1)


def kernel(x, W):
    B, S, H = x.shape
    E = W.shape[0]
    T = 4096
    ns = S // T

    idx_t, w_t, aux = pl.pallas_call(
        _router_kernel,
        grid=(B, ns),
        in_specs=[
            pl.BlockSpec((1, T, H // 2), lambda b, s: (b, s, 0)),
            pl.BlockSpec((1, T, H // 2), lambda b, s: (b, s, 1)),
            pl.BlockSpec((E, H), lambda b, s: (0, 0)),
        ],
        out_specs=[
            pl.BlockSpec((_TOP_K, T), lambda b, s: (0, b * (S // T) + s)),
            pl.BlockSpec((_TOP_K, T), lambda b, s: (0, b * (S // T) + s)),
            pl.BlockSpec((1, 1), lambda b, s: (0, 0)),
        ],
        out_shape=[
            jax.ShapeDtypeStruct((_TOP_K, B * S), jnp.int32),
            jax.ShapeDtypeStruct((_TOP_K, B * S), jnp.float32),
            jax.ShapeDtypeStruct((1, 1), jnp.float32),
        ],
        scratch_shapes=[
            pltpu.VMEM((E, 1), jnp.float32),
            pltpu.VMEM((E, 1), jnp.float32),
        ],
        compiler_params=pltpu.CompilerParams(
            dimension_semantics=("arbitrary", "arbitrary"),
        ),
    )(x, x, W)

    return (
        jnp.moveaxis(idx_t.reshape(_TOP_K, B, S), 0, -1),
        jnp.moveaxis(w_t.reshape(_TOP_K, B, S), 0, -1),
        aux[0, 0],
    )


# speedup vs baseline: 1.1168x; 1.0068x over previous
import jax
import jax.numpy as jnp
from jax.experimental import pallas as pl
from jax.experimental.pallas import tpu as pltpu

_TOP_K = 2


def _router_kernel(x_ref, w_ref, idx_ref, w_out_ref, aux_ref, imp_ref, load_ref):
    b = pl.program_id(0)
    s = pl.program_id(1)
    nb = pl.num_programs(0)
    ns = pl.num_programs(1)

    x2 = x_ref[0]
    logits = jax.lax.dot_general(
        w_ref[:], x2, (((1,), (1,)), ((), ())),
        preferred_element_type=jnp.float32,
    )

    E, T = logits.shape
    row = jax.lax.broadcasted_iota(jnp.int32, (E, T), 0)

    m = jnp.max(logits, axis=0, keepdims=True)
    eq1 = logits == m
    i1 = jnp.min(jnp.where(eq1, row, E), axis=0, keepdims=True)
    sel1 = row == i1
    lmask = jnp.where(sel1, -jnp.inf, logits)
    l2 = jnp.max(lmask, axis=0, keepdims=True)
    eq2 = lmask == l2
    i2 = jnp.min(jnp.where(eq2, row, E), axis=0, keepdims=True)

    e2 = jnp.exp(l2 - m)
    w1 = 1.0 / (1.0 + e2)
    idx_ref[:] = jnp.concatenate([i1, i2], axis=0)
    w_out_ref[:] = jnp.concatenate([w1, e2 * w1], axis=0)

    e = jnp.exp(logits - m)
    z = jnp.sum(e, axis=0, keepdims=True)
    probs = e * (1.0 / z)
    imp_tile = jnp.sum(probs, axis=1, keepdims=True)
    sel = jnp.logical_or(sel1, row == i2)
    load_tile = jnp.sum(sel.astype(jnp.float32), axis=1, keepdims=True)

    first = jnp.logical_and(b == 0, s == 0)
    last = jnp.logical_and(b == nb - 1, s == ns - 1)

    @pl.when(first)
    def _():
        imp_ref[:] = imp_tile
        load_ref[:] = load_tile

    @pl.when(jnp.logical_not(first))
    def _():
        imp_ref[:] = imp_ref[:] + imp_tile
        load_ref[:] = load_ref[:] + load_tile

    @pl.when(last)
    def _():
        n_tok = jnp.float32(nb * ns * T)
        importance = imp_ref[:] / n_tok
        load = load_ref[:] / (n_tok * _TOP_K)
        aux_ref[:] = jnp.sum(E * importance * load).reshape(1, 1)


def kernel(x, W):
    B, S, H = x.shape
    E = W.shape[0]
    T = 4096
    ns = S // T

    idx_t, w_t, aux = pl.pallas_call(
        _router_kernel,
        grid=(B, ns),
        in_specs=[
            pl.BlockSpec((1, T, H), lambda b, s: (b, s, 0)),
            pl.BlockSpec((E, H), lambda b, s: (0, 0)),
        ],
        out_specs=[
            pl.BlockSpec((_TOP_K, T), lambda b, s: (0, b * (S // T) + s)),
            pl.BlockSpec((_TOP_K, T), lambda b, s: (0, b * (S // T) + s)),
            pl.BlockSpec((1, 1), lambda b, s: (0, 0)),
        ],
        out_shape=[
            jax.ShapeDtypeStruct((_TOP_K, B * S), jnp.int32),
            jax.ShapeDtypeStruct((_TOP_K, B * S), jnp.float32),
            jax.ShapeDtypeStruct((1, 1), jnp.float32),
        ],
        scratch_shapes=[
            pltpu.VMEM((E, 1), jnp.float32),
            pltpu.VMEM((E, 1), jnp.float32),
        ],
        compiler_params=pltpu.CompilerParams(
            dimension_semantics=("arbitrary", "arbitrary"),
        ),
    )(x, W)

    return (
        jnp.moveaxis(idx_t.reshape(_TOP_K, B, S), 0, -1),
        jnp.moveaxis(w_t.reshape(_TOP_K, B, S), 0, -1),
        aux[0, 0],
    )
